# baseline (speedup 1.0000x reference)
import functools

import jax
import jax.numpy as jnp
from jax import lax
from jax.experimental import pallas as pl
from jax.experimental.pallas import tpu as pltpu
from jax.experimental.pallas import tpu_sc as plsc

_NC, _NS = 2, 16
_NW = _NC * _NS


def _sc_gather(idxs, tables, B, D):
    n_feat = len(tables)
    mesh = plsc.ScalarSubcoreMesh(axis_name="c", num_cores=_NC)

    @functools.partial(
        pl.kernel,
        out_type=jax.ShapeDtypeStruct((n_feat * B, D), jnp.float32),
        mesh=mesh,
        scratch_types=[
            pltpu.SMEM((B,), jnp.int32),
            pltpu.SemaphoreType.DMA,
        ],
        compiler_params=pltpu.CompilerParams(use_tc_tiling_on_sc=True),
    )
    def gather_kernel(*refs):
        idx_refs = refs[:n_feat]
        tbl_refs = refs[n_feat:2 * n_feat]
        out_hbm = refs[2 * n_feat]
        idx_s, sem = refs[2 * n_feat + 1:]
        core = lax.axis_index("c")
        for f in range(n_feat):
            tbl = tbl_refs[f]

            @pl.when(core == f % _NC)
            def _(f=f, tbl=tbl):
                pltpu.sync_copy(idx_refs[f], idx_s)

                @pl.loop(0, B)
                def _(i):
                    pltpu.async_copy(
                        tbl.at[pl.ds(idx_s[i], 1)],
                        out_hbm.at[pl.ds(f * B + i, 1)], sem)

                pltpu.make_async_copy(
                    tbl.at[pl.ds(0, B)] if tbl.shape[0] >= B
                    else out_hbm.at[pl.ds(0, B)],
                    out_hbm.at[pl.ds(f * B, B)], sem).wait()

    return gather_kernel(*idxs, *tables)


def _mlp_body(x_ref, w1_ref, b1_ref, w2_ref, b2_ref, w3_ref, b3_ref,
              pid_ref, uid_ref, ww_ref, wb_ref, d_ref, w_ref):
    x = x_ref[...].astype(jnp.bfloat16)
    h = jnp.dot(x, w1_ref[...], preferred_element_type=jnp.float32)
    h = jnp.maximum(h + b1_ref[...], 0.0).astype(jnp.bfloat16)
    h = jnp.dot(h, w2_ref[...], preferred_element_type=jnp.float32)
    h = jnp.maximum(h + b2_ref[...], 0.0).astype(jnp.bfloat16)
    d = jnp.dot(h, w3_ref[...], preferred_element_type=jnp.float32)
    d_ref[...] = d + b3_ref[...]
    ww = ww_ref[...].astype(jnp.bfloat16).astype(jnp.float32)
    pb = pid_ref[...].astype(jnp.bfloat16).astype(jnp.float32)
    ub = uid_ref[...].astype(jnp.bfloat16).astype(jnp.float32)
    w_ref[...] = pb * ww[0:1, 0:1] + ub * ww[0:1, 1:2] + wb_ref[...]


def _writer_body(d_ref, w_ref, o_ref):
    dv = d_ref[...]
    wv = w_ref[...]
    t = dv[:, None, :] + wv[None, :, :]
    o_ref[...] = 0.5 * jnp.tanh(0.5 * t) + 0.5


def kernel(product_id, user_id, year, month, day_of_week, hour,
           product_table, user_table, year_table, month_table, dow_table,
           hour_table, wide_W, wide_b, W1, b1, W2, b2, W3, b3):
    B = product_id.shape[0]
    D = product_table.shape[1]
    n_feat = 6

    idxs = [a.reshape(-1).astype(jnp.int32)
            for a in (product_id, user_id, year, month, day_of_week, hour)]
    tables = (product_table, user_table, year_table, month_table, dow_table,
              hour_table)

    deep_in = _sc_gather(idxs, tables, B, D)

    d, w = pl.pallas_call(
        _mlp_body,
        out_shape=(
            jax.ShapeDtypeStruct((n_feat * B, W3.shape[0]), jnp.float32),
            jax.ShapeDtypeStruct((B, 1), jnp.float32),
        ),
    )(deep_in, W1.T.astype(jnp.bfloat16), b1.reshape(1, -1),
      W2.T.astype(jnp.bfloat16), b2.reshape(1, -1),
      W3.T.astype(jnp.bfloat16), b3.reshape(1, -1),
      product_id.astype(jnp.float32),
      user_id.astype(jnp.float32), wide_W, wide_b.reshape(1, 1))

    BI = 16
    n_out = W3.shape[0]
    out = pl.pallas_call(
        _writer_body,
        grid=(n_feat * B // BI,),
        in_specs=[
            pl.BlockSpec((BI, n_out), lambda i: (i, 0)),
            pl.BlockSpec((B, 1), lambda i: (0, 0)),
        ],
        out_specs=pl.BlockSpec((BI, B, n_out), lambda i: (i, 0, 0)),
        out_shape=jax.ShapeDtypeStruct((n_feat * B, B, n_out), jnp.float32),
    )(d, w)
    return out


# speedup vs baseline: 3.0053x; 3.0053x over previous
import functools

import jax
import jax.numpy as jnp
from jax import lax
from jax.experimental import pallas as pl
from jax.experimental.pallas import tpu as pltpu
from jax.experimental.pallas import tpu_sc as plsc

_NC = 2


def _tc_gather_body(c0_ref, oh_ref, pt_ref, ut_ref, x_ref, win, sem):
    n_idx = x_ref.shape[0]
    half = n_idx // 2

    def issue(i, tbl):
        c0 = pl.multiple_of(c0_ref[i], 128)
        pltpu.make_async_copy(
            tbl.at[:, pl.ds(c0, 128)], win.at[i], sem).start()

    lax.fori_loop(0, half, lambda i, _: (issue(i, pt_ref), 0)[1], 0)
    lax.fori_loop(half, n_idx, lambda i, _: (issue(i, ut_ref), 0)[1], 0)

    def drain(i, _):
        pltpu.make_async_copy(
            pt_ref.at[:, pl.ds(0, 128)], win.at[i], sem).wait()
        return 0

    lax.fori_loop(0, n_idx, drain, 0)
    x_ref[...] = jnp.sum(win[...] * oh_ref[...][:, None, :], axis=2)


def _tc_gather(c0_all, oh_all, pt_t, ut_t, D):
    n_idx = c0_all.shape[0]
    return pl.pallas_call(
        _tc_gather_body,
        in_specs=[
            pl.BlockSpec(memory_space=pltpu.SMEM),
            pl.BlockSpec(memory_space=pltpu.VMEM),
            pl.BlockSpec(memory_space=pl.ANY),
            pl.BlockSpec(memory_space=pl.ANY),
        ],
        out_shape=jax.ShapeDtypeStruct((n_idx, D), jnp.float32),
        scratch_shapes=[
            pltpu.VMEM((n_idx, D, 128), jnp.float32),
            pltpu.SemaphoreType.DMA,
        ],
    )(c0_all, oh_all, pt_t, ut_t)


def _sc_calendar_gather(idxs, tables, B, D):
    n_feat = len(tables)
    mesh = plsc.ScalarSubcoreMesh(axis_name="c", num_cores=_NC)

    @functools.partial(
        pl.kernel,
        out_type=jax.ShapeDtypeStruct((n_feat * B, D), jnp.float32),
        mesh=mesh,
        scratch_types=[
            pltpu.SMEM((B,), jnp.int32),
            pltpu.SemaphoreType.DMA,
        ],
    )
    def cal_kernel(*refs):
        idx_refs = refs[:n_feat]
        tbl_refs = refs[n_feat:2 * n_feat]
        out_hbm = refs[2 * n_feat]
        idx_s, sem = refs[2 * n_feat + 1:]
        core = lax.axis_index("c")
        for f in range(n_feat):
            tbl = tbl_refs[f]

            @pl.when(core == f % _NC)
            def _(f=f, tbl=tbl):
                pltpu.sync_copy(idx_refs[f], idx_s)

                @pl.loop(0, B)
                def _(i):
                    pltpu.async_copy(
                        tbl.at[pl.ds(idx_s[i], 1)],
                        out_hbm.at[pl.ds(f * B + i, 1)], sem)

                pltpu.make_async_copy(
                    tbl.at[pl.ds(0, B)] if tbl.shape[0] >= B
                    else out_hbm.at[pl.ds(0, B)],
                    out_hbm.at[pl.ds(f * B, B)], sem).wait()

    return cal_kernel(*idxs, *tables)


def _mlp_body(xpu_ref, xcal_ref, tp_tail_ref, tu_tail_ref, itp_ref, itu_ref,
              w1_ref, b1_ref, w2_ref, b2_ref, w3_ref, b3_ref,
              pid_ref, uid_ref, ww_ref, wb_ref, d_ref, wm_ref):
    def tiny_lookup(tbl_ref, idx):
        V = tbl_ref.shape[0]
        oh = (lax.broadcasted_iota(jnp.int32, (1, V), 1)
              == idx).astype(jnp.bfloat16)
        return jnp.dot(oh, tbl_ref[...].astype(jnp.bfloat16),
                       preferred_element_type=jnp.float32)

    B = itp_ref.shape[0]
    xpu = xpu_ref[...]
    x_p = jnp.where(itp_ref[...] >= 0, tiny_lookup(tp_tail_ref, itp_ref[...]),
                    xpu[:B])
    x_u = jnp.where(itu_ref[...] >= 0, tiny_lookup(tu_tail_ref, itu_ref[...]),
                    xpu[B:])
    x = jnp.concatenate([x_p, x_u, xcal_ref[...]],
                        axis=0).astype(jnp.bfloat16)

    h = jnp.dot(x, w1_ref[...], preferred_element_type=jnp.float32)
    h = jnp.maximum(h + b1_ref[...], 0.0).astype(jnp.bfloat16)
    h = jnp.dot(h, w2_ref[...], preferred_element_type=jnp.float32)
    h = jnp.maximum(h + b2_ref[...], 0.0).astype(jnp.bfloat16)
    d = jnp.dot(h, w3_ref[...], preferred_element_type=jnp.float32)
    d_ref[...] = 0.5 * (d + b3_ref[...])

    ww = ww_ref[...].astype(jnp.bfloat16).astype(jnp.float32)
    pb = pid_ref[...].astype(jnp.bfloat16).astype(jnp.float32)
    ub = uid_ref[...].astype(jnp.bfloat16).astype(jnp.float32)
    wcol = pb * ww[0:1, 0:1] + ub * ww[0:1, 1:2] + wb_ref[...]
    wm_ref[...] = 0.5 * jnp.broadcast_to(wcol, wm_ref.shape)


def _writer_body(d_ref, wm_ref, o_ref):
    t = d_ref[...][:, None, :] + wm_ref[...][None, :, :]
    o_ref[...] = 0.5 * jnp.tanh(t) + 0.5


def kernel(product_id, user_id, year, month, day_of_week, hour,
           product_table, user_table, year_table, month_table, dow_table,
           hour_table, wide_W, wide_b, W1, b1, W2, b2, W3, b3):
    B = product_id.shape[0]
    D = product_table.shape[1]

    pid = product_id.reshape(-1).astype(jnp.int32)
    uid = user_id.reshape(-1).astype(jnp.int32)
    VP, VU = product_table.shape[0], user_table.shape[0]
    FP, FU = (VP // 128) * 128, (VU // 128) * 128
    c0_p = jnp.minimum(pid & ~127, FP - 128)
    c0_u = jnp.minimum(uid & ~127, FU - 128)
    rem = jnp.concatenate([jnp.clip(pid - c0_p, 0, 127),
                           jnp.clip(uid - c0_u, 0, 127)])
    oh_all = (jnp.arange(128, dtype=jnp.int32)[None, :]
              == rem[:, None]).astype(jnp.float32)
    c0_all = jnp.concatenate([c0_p, c0_u])
    it_p = (pid - FP).reshape(B, 1)
    it_u = (uid - FU).reshape(B, 1)

    x_pu = _tc_gather(c0_all, oh_all, product_table.T, user_table.T, D)

    cal_idx = [a.reshape(-1).astype(jnp.int32)
               for a in (year, month, day_of_week, hour)]
    x_cal = _sc_calendar_gather(
        cal_idx, (year_table, month_table, dow_table, hour_table), B, D)

    n_out = W3.shape[0]
    d2, wm = pl.pallas_call(
        _mlp_body,
        out_shape=(
            jax.ShapeDtypeStruct((6 * B, n_out), jnp.float32),
            jax.ShapeDtypeStruct((B, n_out), jnp.float32),
        ),
    )(x_pu, x_cal, product_table[FP:], user_table[FU:], it_p, it_u,
      W1.T.astype(jnp.bfloat16), b1.reshape(1, -1),
      W2.T.astype(jnp.bfloat16), b2.reshape(1, -1),
      W3.T.astype(jnp.bfloat16), b3.reshape(1, -1),
      product_id.astype(jnp.float32), user_id.astype(jnp.float32),
      wide_W, wide_b.reshape(1, 1))

    BI = 16
    out = pl.pallas_call(
        _writer_body,
        grid=(6 * B // BI,),
        in_specs=[
            pl.BlockSpec((BI, n_out), lambda i: (i, 0)),
            pl.BlockSpec((B, n_out), lambda i: (0, 0)),
        ],
        out_specs=pl.BlockSpec((BI, B, n_out), lambda i: (i, 0, 0)),
        out_shape=jax.ShapeDtypeStruct((6 * B, B, n_out), jnp.float32),
    )(d2, wm)
    return out


# speedup vs baseline: 3.0627x; 1.0191x over previous
import functools

import jax
import jax.numpy as jnp
from jax import lax
from jax.experimental import pallas as pl
from jax.experimental.pallas import tpu as pltpu
from jax.experimental.pallas import tpu_sc as plsc

_NC = 2


def _tc_gather_body(c0_ref, rem_ref, pt_ref, ut_ref, x_ref, win, sem):
    n_idx = x_ref.shape[0]
    half = n_idx // 2

    def issue(i, tbl):
        c0 = pl.multiple_of(c0_ref[i], 128)
        pltpu.make_async_copy(
            tbl.at[:, pl.ds(c0, 128)], win.at[i], sem).start()

    lax.fori_loop(0, half, lambda i, _: (issue(i, pt_ref), 0)[1], 0)
    lax.fori_loop(half, n_idx, lambda i, _: (issue(i, ut_ref), 0)[1], 0)

    def drain(i, _):
        pltpu.make_async_copy(
            pt_ref.at[:, pl.ds(0, 128)], win.at[i], sem).wait()
        return 0

    lax.fori_loop(0, n_idx, drain, 0)
    oh = (lax.broadcasted_iota(jnp.int32, (n_idx, 128), 1)
          == rem_ref[...]).astype(jnp.float32)
    x_ref[...] = jnp.sum(win[...] * oh[:, None, :], axis=2)


def _tc_gather(c0_all, rem_all, pt_t, ut_t, D):
    n_idx = c0_all.shape[0]
    return pl.pallas_call(
        _tc_gather_body,
        in_specs=[
            pl.BlockSpec(memory_space=pltpu.SMEM),
            pl.BlockSpec(memory_space=pltpu.VMEM),
            pl.BlockSpec(memory_space=pl.ANY),
            pl.BlockSpec(memory_space=pl.ANY),
        ],
        out_shape=jax.ShapeDtypeStruct((n_idx, D), jnp.float32),
        scratch_shapes=[
            pltpu.VMEM((n_idx, D, 128), jnp.float32),
            pltpu.SemaphoreType.DMA,
        ],
    )(c0_all, rem_all, pt_t, ut_t)


def _sc_calendar_gather(idxs, tables, B, D):
    n_feat = len(tables)
    mesh = plsc.ScalarSubcoreMesh(axis_name="c", num_cores=_NC)

    @functools.partial(
        pl.kernel,
        out_type=jax.ShapeDtypeStruct((n_feat * B, D), jnp.float32),
        mesh=mesh,
        scratch_types=[
            pltpu.SMEM((B,), jnp.int32),
            pltpu.SemaphoreType.DMA,
        ],
    )
    def cal_kernel(*refs):
        idx_refs = refs[:n_feat]
        tbl_refs = refs[n_feat:2 * n_feat]
        out_hbm = refs[2 * n_feat]
        idx_s, sem = refs[2 * n_feat + 1:]
        core = lax.axis_index("c")
        for f in range(n_feat):
            tbl = tbl_refs[f]

            @pl.when(core == f % _NC)
            def _(f=f, tbl=tbl):
                pltpu.sync_copy(idx_refs[f], idx_s)

                @pl.loop(0, B)
                def _(i):
                    pltpu.async_copy(
                        tbl.at[pl.ds(idx_s[i], 1)],
                        out_hbm.at[pl.ds(f * B + i, 1)], sem)

                pltpu.make_async_copy(
                    tbl.at[pl.ds(0, B)] if tbl.shape[0] >= B
                    else out_hbm.at[pl.ds(0, B)],
                    out_hbm.at[pl.ds(f * B, B)], sem).wait()

    return cal_kernel(*idxs, *tables)


def _mlp_body(xpu_ref, xcal_ref, tp_tail_ref, tu_tail_ref, itp_ref, itu_ref,
              w1_ref, b1_ref, w2_ref, b2_ref, w3_ref, b3_ref,
              pid_ref, uid_ref, ww_ref, wb_ref, d_ref, wm_ref):
    def tiny_lookup(tbl_ref, idx):
        V = tbl_ref.shape[0]
        oh = (lax.broadcasted_iota(jnp.int32, (1, V), 1)
              == idx).astype(jnp.bfloat16)
        return jnp.dot(oh, tbl_ref[...].astype(jnp.bfloat16),
                       preferred_element_type=jnp.float32)

    B = itp_ref.shape[0]
    xpu = xpu_ref[...]
    x_p = jnp.where(itp_ref[...] >= 0, tiny_lookup(tp_tail_ref, itp_ref[...]),
                    xpu[:B])
    x_u = jnp.where(itu_ref[...] >= 0, tiny_lookup(tu_tail_ref, itu_ref[...]),
                    xpu[B:])
    x = jnp.concatenate([x_p, x_u, xcal_ref[...]],
                        axis=0).astype(jnp.bfloat16)

    nt = (((1,), (1,)), ((), ()))
    h = lax.dot_general(x, w1_ref[...].astype(jnp.bfloat16), nt,
                        preferred_element_type=jnp.float32)
    h = jnp.maximum(h + b1_ref[...], 0.0).astype(jnp.bfloat16)
    h = lax.dot_general(h, w2_ref[...].astype(jnp.bfloat16), nt,
                        preferred_element_type=jnp.float32)
    h = jnp.maximum(h + b2_ref[...], 0.0).astype(jnp.bfloat16)
    d = lax.dot_general(h, w3_ref[...].astype(jnp.bfloat16), nt,
                        preferred_element_type=jnp.float32)
    d_ref[...] = 0.5 * (d + b3_ref[...])

    ww = ww_ref[...].astype(jnp.bfloat16).astype(jnp.float32)
    pb = pid_ref[...].astype(jnp.bfloat16).astype(jnp.float32)
    ub = uid_ref[...].astype(jnp.bfloat16).astype(jnp.float32)
    wcol = pb * ww[0:1, 0:1] + ub * ww[0:1, 1:2] + wb_ref[...]
    wm_ref[...] = 0.5 * jnp.broadcast_to(wcol, wm_ref.shape)


def _writer_body(d_ref, wm_ref, o_ref):
    t = d_ref[...][:, None, :] + wm_ref[...][None, :, :]
    o_ref[...] = 0.5 * jnp.tanh(t) + 0.5


def kernel(product_id, user_id, year, month, day_of_week, hour,
           product_table, user_table, year_table, month_table, dow_table,
           hour_table, wide_W, wide_b, W1, b1, W2, b2, W3, b3):
    B = product_id.shape[0]
    D = product_table.shape[1]

    pid = product_id.reshape(-1).astype(jnp.int32)
    uid = user_id.reshape(-1).astype(jnp.int32)
    VP, VU = product_table.shape[0], user_table.shape[0]
    FP, FU = (VP // 128) * 128, (VU // 128) * 128
    c0_p = jnp.minimum(pid & ~127, FP - 128)
    c0_u = jnp.minimum(uid & ~127, FU - 128)
    rem_all = jnp.concatenate([jnp.clip(pid - c0_p, 0, 127),
                               jnp.clip(uid - c0_u, 0, 127)]).reshape(-1, 1)
    c0_all = jnp.concatenate([c0_p, c0_u])
    it_p = (pid - FP).reshape(B, 1)
    it_u = (uid - FU).reshape(B, 1)

    x_pu = _tc_gather(c0_all, rem_all, product_table.T, user_table.T, D)

    cal_idx = [a.reshape(-1).astype(jnp.int32)
               for a in (year, month, day_of_week, hour)]
    x_cal = _sc_calendar_gather(
        cal_idx, (year_table, month_table, dow_table, hour_table), B, D)

    n_out = W3.shape[0]
    d2, wm = pl.pallas_call(
        _mlp_body,
        out_shape=(
            jax.ShapeDtypeStruct((6 * B, n_out), jnp.float32),
            jax.ShapeDtypeStruct((B, n_out), jnp.float32),
        ),
    )(x_pu, x_cal, product_table[FP:], user_table[FU:], it_p, it_u,
      W1, b1.reshape(1, -1), W2, b2.reshape(1, -1), W3, b3.reshape(1, -1),
      product_id.astype(jnp.float32), user_id.astype(jnp.float32),
      wide_W, wide_b.reshape(1, 1))

    BI = 16
    out = pl.pallas_call(
        _writer_body,
        grid=(6 * B // BI,),
        in_specs=[
            pl.BlockSpec((BI, n_out), lambda i: (i, 0)),
            pl.BlockSpec((B, n_out), lambda i: (0, 0)),
        ],
        out_specs=pl.BlockSpec((BI, B, n_out), lambda i: (i, 0, 0)),
        out_shape=jax.ShapeDtypeStruct((6 * B, B, n_out), jnp.float32),
    )(d2, wm)
    return out


# speedup vs baseline: 3.1395x; 1.0251x over previous
import functools

import jax
import jax.numpy as jnp
from jax import lax
from jax.experimental import pallas as pl
from jax.experimental.pallas import tpu as pltpu
from jax.experimental.pallas import tpu_sc as plsc

_NC = 2


def _tc_gather_body(c0_ref, rem_ref, pt_ref, ut_ref, x_ref, win, sem):
    n_idx = x_ref.shape[0]
    half = n_idx // 2

    def issue(i, tbl):
        c0 = pl.multiple_of(c0_ref[i], 128)
        pltpu.make_async_copy(
            tbl.at[:, pl.ds(c0, 128)], win.at[i], sem).start()

    lax.fori_loop(0, half, lambda i, _: (issue(i, pt_ref), 0)[1], 0)
    lax.fori_loop(half, n_idx, lambda i, _: (issue(i, ut_ref), 0)[1], 0)

    def drain(i, _):
        pltpu.make_async_copy(
            pt_ref.at[:, pl.ds(0, 128)], win.at[i], sem).wait()
        return 0

    lax.fori_loop(0, n_idx, drain, 0)
    oh = (lax.broadcasted_iota(jnp.int32, (n_idx, 128), 1)
          == rem_ref[...]).astype(jnp.float32)
    x_ref[...] = jnp.sum(win[...] * oh[:, None, :], axis=2)


def _tc_gather(c0_all, rem_all, pt_t, ut_t, D):
    n_idx = c0_all.shape[0]
    return pl.pallas_call(
        _tc_gather_body,
        in_specs=[
            pl.BlockSpec(memory_space=pltpu.SMEM),
            pl.BlockSpec(memory_space=pltpu.VMEM),
            pl.BlockSpec(memory_space=pl.ANY),
            pl.BlockSpec(memory_space=pl.ANY),
        ],
        out_shape=jax.ShapeDtypeStruct((n_idx, D), jnp.float32),
        scratch_shapes=[
            pltpu.VMEM((n_idx, D, 128), jnp.float32),
            pltpu.SemaphoreType.DMA,
        ],
    )(c0_all, rem_all, pt_t, ut_t)


def _sc_calendar_gather(idxs, tables, B, D):
    n_feat = len(tables)
    mesh = plsc.ScalarSubcoreMesh(axis_name="c", num_cores=_NC)

    @functools.partial(
        pl.kernel,
        out_type=jax.ShapeDtypeStruct((n_feat * B, D), jnp.float32),
        mesh=mesh,
        scratch_types=[
            pltpu.SMEM((B,), jnp.int32),
            pltpu.SemaphoreType.DMA,
        ],
    )
    def cal_kernel(*refs):
        idx_refs = refs[:n_feat]
        tbl_refs = refs[n_feat:2 * n_feat]
        out_hbm = refs[2 * n_feat]
        idx_s, sem = refs[2 * n_feat + 1:]
        core = lax.axis_index("c")
        for f in range(n_feat):
            tbl = tbl_refs[f]

            @pl.when(core == f % _NC)
            def _(f=f, tbl=tbl):
                pltpu.sync_copy(idx_refs[f], idx_s)

                @pl.loop(0, B)
                def _(i):
                    pltpu.async_copy(
                        tbl.at[pl.ds(idx_s[i], 1)],
                        out_hbm.at[pl.ds(f * B + i, 1)], sem)

                pltpu.make_async_copy(
                    tbl.at[pl.ds(0, B)] if tbl.shape[0] >= B
                    else out_hbm.at[pl.ds(0, B)],
                    out_hbm.at[pl.ds(f * B, B)], sem).wait()

    return cal_kernel(*idxs, *tables)


def _mlp_body(xpu_ref, xcal_ref, tp_tail_ref, tu_tail_ref, itp_ref, itu_ref,
              w1_ref, b1_ref, w2_ref, b2_ref, w3_ref, b3_ref,
              pid_ref, uid_ref, ww_ref, wb_ref, d_ref, wm_ref):
    def tiny_lookup(tbl_ref, idx):
        V = tbl_ref.shape[0]
        oh = (lax.broadcasted_iota(jnp.int32, (1, V), 1)
              == idx).astype(jnp.bfloat16)
        return jnp.dot(oh, tbl_ref[...].astype(jnp.bfloat16),
                       preferred_element_type=jnp.float32)

    B = itp_ref.shape[0]
    xpu = xpu_ref[...]
    x_p = jnp.where(itp_ref[...] >= 0, tiny_lookup(tp_tail_ref, itp_ref[...]),
                    xpu[:B])
    x_u = jnp.where(itu_ref[...] >= 0, tiny_lookup(tu_tail_ref, itu_ref[...]),
                    xpu[B:])
    x = jnp.concatenate([x_p, x_u, xcal_ref[...]],
                        axis=0).astype(jnp.bfloat16)

    nt = (((1,), (1,)), ((), ()))
    h = lax.dot_general(x, w1_ref[...].astype(jnp.bfloat16), nt,
                        preferred_element_type=jnp.float32)
    h = jnp.maximum(h + b1_ref[...], 0.0).astype(jnp.bfloat16)
    h = lax.dot_general(h, w2_ref[...].astype(jnp.bfloat16), nt,
                        preferred_element_type=jnp.float32)
    h = jnp.maximum(h + b2_ref[...], 0.0).astype(jnp.bfloat16)
    d = lax.dot_general(h, w3_ref[...].astype(jnp.bfloat16), nt,
                        preferred_element_type=jnp.float32)
    d_ref[...] = 0.5 * (d + b3_ref[...])

    ww = ww_ref[...].astype(jnp.bfloat16).astype(jnp.float32)
    pb = pid_ref[...].astype(jnp.bfloat16).astype(jnp.float32)
    ub = uid_ref[...].astype(jnp.bfloat16).astype(jnp.float32)
    wcol = pb * ww[0:1, 0:1] + ub * ww[0:1, 1:2] + wb_ref[...]
    wm_ref[...] = 0.5 * jnp.broadcast_to(wcol, wm_ref.shape)


def _writer_body(d_ref, wm_ref, o_ref):
    t = d_ref[...][:, None, :] + wm_ref[...][None, :, :]
    o_ref[...] = 0.5 * jnp.tanh(t) + 0.5


def kernel(product_id, user_id, year, month, day_of_week, hour,
           product_table, user_table, year_table, month_table, dow_table,
           hour_table, wide_W, wide_b, W1, b1, W2, b2, W3, b3):
    B = product_id.shape[0]
    D = product_table.shape[1]

    pid = product_id.reshape(-1).astype(jnp.int32)
    uid = user_id.reshape(-1).astype(jnp.int32)
    VP, VU = product_table.shape[0], user_table.shape[0]
    FP, FU = (VP // 128) * 128, (VU // 128) * 128
    c0_p = jnp.minimum(pid & ~127, FP - 128)
    c0_u = jnp.minimum(uid & ~127, FU - 128)
    rem_all = jnp.concatenate([jnp.clip(pid - c0_p, 0, 127),
                               jnp.clip(uid - c0_u, 0, 127)]).reshape(-1, 1)
    c0_all = jnp.concatenate([c0_p, c0_u])
    it_p = (pid - FP).reshape(B, 1)
    it_u = (uid - FU).reshape(B, 1)

    x_pu = _tc_gather(c0_all, rem_all, product_table.T, user_table.T, D)

    cal_idx = [a.reshape(-1).astype(jnp.int32)
               for a in (year, month, day_of_week, hour)]
    x_cal = _sc_calendar_gather(
        cal_idx, (year_table, month_table, dow_table, hour_table), B, D)

    n_out = W3.shape[0]
    d2, wm = pl.pallas_call(
        _mlp_body,
        out_shape=(
            jax.ShapeDtypeStruct((6 * B, n_out), jnp.float32),
            jax.ShapeDtypeStruct((B, n_out), jnp.float32),
        ),
    )(x_pu, x_cal, product_table[FP:], user_table[FU:], it_p, it_u,
      W1, b1.reshape(1, -1), W2, b2.reshape(1, -1), W3, b3.reshape(1, -1),
      product_id.astype(jnp.float32), user_id.astype(jnp.float32),
      wide_W, wide_b.reshape(1, 1))

    BI = 32
    out = pl.pallas_call(
        _writer_body,
        grid=(6 * B // BI,),
        in_specs=[
            pl.BlockSpec((BI, n_out), lambda i: (i, 0)),
            pl.BlockSpec((B, n_out), lambda i: (0, 0)),
        ],
        out_specs=pl.BlockSpec((BI, B, n_out), lambda i: (i, 0, 0)),
        out_shape=jax.ShapeDtypeStruct((6 * B, B, n_out), jnp.float32),
    )(d2, wm)
    return out
